# baseline (speedup 1.0000x reference)
import functools

import jax
import jax.numpy as jnp
from jax import lax
from jax.experimental import pallas as pl
from jax.experimental.pallas import tpu as pltpu
from jax.experimental.pallas import tpu_sc as plsc

_NEG_TH = 0.3
_MSE_TH = 1.0
_K = 8
_L = 16
_NW = 32


def _sc_body(rows_per_worker, num_chunks, wid_fn, pred_hbm, lab_hbm, out_hbm,
             pbuf, lbuf, obuf):
    wid = wid_fn()
    base_row = wid * rows_per_worker
    lanes = lax.iota(jnp.int32, _L)
    big_i32 = jnp.full((_L,), jnp.int32(2**31 - 1))

    def row_body(r, accs):
        acc_top8, acc_pv, acc_valid, acc_sq = accs
        row = base_row + r
        pltpu.sync_copy(pred_hbm.at[row], pbuf)
        pltpu.sync_copy(lab_hbm.at[row], lbuf)

        def chunk_body(c, carry):
            top16, lmax, lidx, lpred, asq = carry
            sl = pl.ds(pl.multiple_of(c * _L, _L), _L)
            p = pbuf[sl]
            l = lbuf[sl]
            neg = l < _NEG_TH
            y = jnp.where(neg, jnp.maximum(p, 0.0), 0.0)
            cdesc = lax.rev(jnp.sort(y), (0,))
            top16 = jnp.sort(jnp.maximum(top16, cdesc))
            idxv = lanes + c * _L
            gt = l > lmax
            lmax = jnp.where(gt, l, lmax)
            lidx = jnp.where(gt, idxv, lidx)
            lpred = jnp.where(gt, p, lpred)
            res = p - l
            rm = jnp.where((jnp.abs(res) > _MSE_TH) & neg, res, 0.0)
            asq = asq + rm * rm
            return top16, lmax, lidx, lpred, asq

        top16_0 = jnp.zeros((_L,), jnp.float32)
        lmax0 = jnp.full((_L,), -jnp.inf, jnp.float32)
        lidx0 = jnp.zeros((_L,), jnp.int32)
        lpred0 = jnp.zeros((_L,), jnp.float32)
        top16, lmax, lidx, lpred, acc_sq = lax.fori_loop(
            0, num_chunks, chunk_body, (top16_0, lmax0, lidx0, lpred0, acc_sq))

        acc_top8 = acc_top8 + jnp.where(lanes >= _L - _K, top16, 0.0)

        m = jnp.max(lmax)
        mv = jnp.full((_L,), 0.0) + m
        is_m = lmax == mv
        ridx = jnp.min(jnp.where(is_m, lidx, big_i32))
        ridxv = jnp.full((_L,), 0) + ridx
        sel = is_m & (lidx == ridxv)
        pa = jnp.max(jnp.where(sel, lpred, -jnp.inf))
        pav = jnp.full((_L,), 0.0) + pa
        valid_v = (mv > _NEG_TH) & (lanes == 0)
        acc_pv = acc_pv + jnp.where(valid_v, jnp.minimum(pav, 1.0), 0.0)
        acc_valid = acc_valid + jnp.where(valid_v, 1.0, 0.0)
        return acc_top8, acc_pv, acc_valid, acc_sq

    z = jnp.zeros((_L,), jnp.float32)
    acc_top8, acc_pv, acc_valid, acc_sq = lax.fori_loop(
        0, rows_per_worker, row_body, (z, z, z, z))

    s_top8 = jnp.full((_L,), 0.0) + jnp.sum(acc_top8)
    s_pv = jnp.full((_L,), 0.0) + jnp.sum(acc_pv)
    s_valid = jnp.full((_L,), 0.0) + jnp.sum(acc_valid)
    s_sq = jnp.full((_L,), 0.0) + jnp.sum(acc_sq)
    outv = jnp.where(lanes == 0, s_top8,
                     jnp.where(lanes == 1, s_pv,
                               jnp.where(lanes == 2, s_valid,
                                         jnp.where(lanes == 3, s_sq, 0.0))))
    obuf[...] = outv
    pltpu.sync_copy(obuf, out_hbm.at[wid])


@functools.partial(jax.jit, static_argnums=(2, 3))
def _run(pred, lab, rows, hw):
    rows_per_worker = rows // _NW
    num_chunks = hw // _L
    mesh = plsc.VectorSubcoreMesh(
        core_axis_name="c", subcore_axis_name="s", num_cores=2, num_subcores=16)
    wid_fn = lambda: lax.axis_index("s") * 2 + lax.axis_index("c")
    body = functools.partial(_sc_body, rows_per_worker, num_chunks, wid_fn)
    parts = pl.kernel(
        body,
        out_type=jax.ShapeDtypeStruct((_NW, _L), jnp.float32),
        mesh=mesh,
        scratch_types=[
            pltpu.VMEM((hw,), jnp.float32),
            pltpu.VMEM((hw,), jnp.float32),
            pltpu.VMEM((_L,), jnp.float32),
        ],
        compiler_params=pltpu.CompilerParams(needs_layout_passes=False),
    )(pred, lab)
    s = jnp.sum(parts, axis=0)
    total_top8, total_pv, total_valid, total_sq = s[0], s[1], s[2], s[3]
    n_valid = jnp.maximum(total_valid, 1.0)
    margin = 1.0 - total_pv / n_valid + total_top8 / (rows * _K)
    mse = total_sq / (rows * hw)
    return margin + mse


def kernel(prediction, label):
    rows = prediction.shape[0]
    hw = prediction.shape[-2] * prediction.shape[-1]
    pred = prediction.reshape(rows, hw)
    lab = label.reshape(rows, hw)
    assert rows % _NW == 0 and hw % _L == 0
    return _run(pred, lab, rows, hw)
